# baseline (speedup 1.0000x reference)
import functools

import jax
import jax.numpy as jnp
from jax import lax
from jax.experimental import pallas as pl
from jax.experimental.pallas import tpu as pltpu
from jax.experimental.pallas import tpu_sc as plsc

NC = 2
NS = 16
CHUNK = 128


def _hist_body(nodes_pad, nch, dstp, deg, dst_v, ones_v, cnt_v, hist, sem):
    del sem
    c = lax.axis_index("c")
    s = lax.axis_index("s")
    rows = nodes_pad // NS

    def init_ones(i, _):
        ones_v[pl.ds(i * 16, 16)] = jnp.ones((16,), jnp.float32)
        return 0

    def init_zero(i, _):
        cnt_v[pl.ds(i * 16, 16)] = jnp.zeros((16,), jnp.float32)
        return 0

    lax.fori_loop(0, CHUNK // 16, init_ones, 0)
    lax.fori_loop(0, rows // 16, init_zero, 0)
    pltpu.sync_copy(cnt_v, hist.at[pl.ds(s * rows, rows)])
    plsc.subcore_barrier()

    pltpu.sync_copy(dstp.at[s], dst_v)

    def scatter(j, _):
        pltpu.sync_copy(ones_v, hist.at[dst_v.at[j]], add=True)
        return 0

    lax.fori_loop(0, nch, scatter, 0)
    plsc.subcore_barrier()

    @pl.when(c == 0)
    def _drain():
        pltpu.sync_copy(hist.at[pl.ds(s * rows, rows)], cnt_v)
        pltpu.sync_copy(cnt_v, deg.at[pl.ds(s * rows, rows)])


def _degree_hist(dstp, nodes_pad):
    ns, nch, _ = dstp.shape
    assert ns == NS
    rows = nodes_pad // NS
    mesh = plsc.VectorSubcoreMesh(core_axis_name="c", subcore_axis_name="s")
    return pl.kernel(
        functools.partial(_hist_body, nodes_pad, nch),
        out_type=jax.ShapeDtypeStruct((nodes_pad,), jnp.float32),
        mesh=mesh,
        scratch_types=[
            pltpu.VMEM((nch, CHUNK), jnp.int32),
            pltpu.VMEM((CHUNK,), jnp.float32),
            pltpu.VMEM((rows,), jnp.float32),
            pltpu.VMEM_SHARED((nodes_pad,), jnp.float32),
            pltpu.SemaphoreType.DMA,
        ],
        name="sc_degree_hist",
    )(dstp)


def _agg_body(n, nch, g, srcp, dstp, out, src_v, dst_v, gbuf, acc, sem):
    del sem
    c = lax.axis_index("c")
    s = lax.axis_index("s")
    col = pl.multiple_of(c * CHUNK, CHUNK)
    rpt = n // NS
    rchunk = CHUNK

    def seed(k, _):
        r0 = s * rpt + k * rchunk
        pltpu.sync_copy(g.at[pl.ds(r0, rchunk), pl.ds(col, CHUNK)], gbuf)
        pltpu.sync_copy(gbuf, acc.at[pl.ds(r0, rchunk)])
        return 0

    lax.fori_loop(0, rpt // rchunk, seed, 0)

    pltpu.sync_copy(srcp.at[s], src_v)
    pltpu.sync_copy(dstp.at[s], dst_v)
    plsc.subcore_barrier()

    def edge_chunk(j, _):
        pltpu.sync_copy(g.at[src_v.at[j], pl.ds(col, CHUNK)], gbuf)
        pltpu.sync_copy(gbuf, acc.at[dst_v.at[j]], add=True)
        return 0

    lax.fori_loop(0, nch, edge_chunk, 0)
    plsc.subcore_barrier()

    def drain(k, _):
        r0 = s * rpt + k * rchunk
        pltpu.sync_copy(acc.at[pl.ds(r0, rchunk)], gbuf)
        pltpu.sync_copy(gbuf, out.at[pl.ds(r0, rchunk), pl.ds(col, CHUNK)])
        return 0

    lax.fori_loop(0, rpt // rchunk, drain, 0)


def _aggregate(g, srcp, dstp):
    np_rows = g.shape[0]
    ns, nch, _ = srcp.shape
    assert ns == NS and np_rows % (NS * CHUNK) == 0
    mesh = plsc.VectorSubcoreMesh(core_axis_name="c", subcore_axis_name="s")
    return pl.kernel(
        functools.partial(_agg_body, np_rows, nch),
        out_type=jax.ShapeDtypeStruct((np_rows, NC * CHUNK), jnp.float32),
        mesh=mesh,
        scratch_types=[
            pltpu.VMEM((nch, CHUNK), jnp.int32),
            pltpu.VMEM((nch, CHUNK), jnp.int32),
            pltpu.VMEM((CHUNK, CHUNK), jnp.float32),
            pltpu.VMEM_SHARED((np_rows, CHUNK), jnp.float32),
            pltpu.SemaphoreType.DMA,
        ],
        name="sc_edge_aggregate",
    )(g, srcp, dstp)


def _mm_body(relu_in, deg_ref, h_ref, w_ref, b_ref, g_ref):
    dis = lax.rsqrt(deg_ref[...] + 1.0)
    h = h_ref[...]
    if relu_in:
        h = jnp.maximum(h * dis, 0.0)
    acc = jnp.dot(h, w_ref[...], preferred_element_type=jnp.float32,
                  precision=lax.Precision.HIGHEST)
    g_ref[...] = (acc + b_ref[...]) * dis


def _matmul_scaled(h, w, b, deg2d, relu_in, block_rows=1024):
    n, d = h.shape
    _, hdim = w.shape
    grid = (n // block_rows,)
    return pl.pallas_call(
        functools.partial(_mm_body, relu_in),
        grid=grid,
        in_specs=[
            pl.BlockSpec((block_rows, 1), lambda i: (i, 0)),
            pl.BlockSpec((block_rows, d), lambda i: (i, 0)),
            pl.BlockSpec((d, hdim), lambda i: (0, 0)),
            pl.BlockSpec((1, hdim), lambda i: (0, 0)),
        ],
        out_specs=pl.BlockSpec((block_rows, hdim), lambda i: (i, 0)),
        out_shape=jax.ShapeDtypeStruct((n, hdim), jnp.float32),
        name="tc_matmul_scaled",
    )(deg2d, h, w, b.reshape(1, hdim))


def _relu_body(deg_ref, a_ref, o_ref):
    dis = lax.rsqrt(deg_ref[...] + 1.0)
    o_ref[...] = jnp.maximum(a_ref[...] * dis, 0.0)


def _relu_scale(a, deg2d, block_rows=1024):
    n, hdim = a.shape
    return pl.pallas_call(
        _relu_body,
        grid=(n // block_rows,),
        in_specs=[
            pl.BlockSpec((block_rows, 1), lambda i: (i, 0)),
            pl.BlockSpec((block_rows, hdim), lambda i: (i, 0)),
        ],
        out_specs=pl.BlockSpec((block_rows, hdim), lambda i: (i, 0)),
        out_shape=jax.ShapeDtypeStruct((n, hdim), jnp.float32),
        name="tc_relu_scale",
    )(deg2d, a)


def kernel(x, edge_index, W1, b1, W2, b2):
    n, d = x.shape
    e = edge_index.shape[1]
    ept = e // NS
    nch = pl.cdiv(ept, CHUNK)
    ept_pad = nch * CHUNK
    np_rows = pl.cdiv(n, NS * CHUNK) * NS * CHUNK

    src = edge_index[0].reshape(NS, ept)
    dst = edge_index[1].reshape(NS, ept)
    pad = ((0, 0), (0, ept_pad - ept))
    srcp = jnp.pad(src, pad).reshape(NS, nch, CHUNK)
    dstp = jnp.pad(dst, pad, constant_values=n).reshape(NS, nch, CHUNK)

    deg_raw = _degree_hist(dstp, np_rows)
    deg2d = deg_raw.reshape(np_rows, 1)
    xp = jnp.pad(x, ((0, np_rows - n), (0, 0)))

    g1 = _matmul_scaled(xp, W1, b1, deg2d, relu_in=False)
    a1 = _aggregate(g1, srcp, dstp)
    g2 = _matmul_scaled(a1, W2, b2, deg2d, relu_in=True)
    a2 = _aggregate(g2, srcp, dstp)
    return _relu_scale(a2, deg2d)[:n]


# speedup vs baseline: 1.0257x; 1.0257x over previous
import functools

import jax
import jax.numpy as jnp
from jax import lax
from jax.experimental import pallas as pl
from jax.experimental.pallas import tpu as pltpu
from jax.experimental.pallas import tpu_sc as plsc

NC = 2
NS = 16
CHUNK = 128


def _hist_body(nodes_pad, nch, dstp, deg, dst_v, ones_v, cnt_v, hist, sem):
    del sem
    c = lax.axis_index("c")
    s = lax.axis_index("s")
    rows = nodes_pad // NS

    def init_ones(i, _):
        ones_v[pl.ds(i * 16, 16)] = jnp.ones((16,), jnp.float32)
        return 0

    def init_zero(i, _):
        cnt_v[pl.ds(i * 16, 16)] = jnp.zeros((16,), jnp.float32)
        return 0

    lax.fori_loop(0, CHUNK // 16, init_ones, 0)
    lax.fori_loop(0, rows // 16, init_zero, 0)
    pltpu.sync_copy(cnt_v, hist.at[pl.ds(s * rows, rows)])
    plsc.subcore_barrier()

    pltpu.sync_copy(dstp.at[s], dst_v)

    def scatter(j, _):
        pltpu.sync_copy(ones_v, hist.at[dst_v.at[j]], add=True)
        return 0

    lax.fori_loop(0, nch, scatter, 0)
    plsc.subcore_barrier()

    @pl.when(c == 0)
    def _drain():
        pltpu.sync_copy(hist.at[pl.ds(s * rows, rows)], cnt_v)
        pltpu.sync_copy(cnt_v, deg.at[pl.ds(s * rows, rows)])


def _degree_hist(dstp, nodes_pad):
    ns, nch, _ = dstp.shape
    assert ns == NS
    rows = nodes_pad // NS
    mesh = plsc.VectorSubcoreMesh(core_axis_name="c", subcore_axis_name="s")
    return pl.kernel(
        functools.partial(_hist_body, nodes_pad, nch),
        out_type=jax.ShapeDtypeStruct((nodes_pad,), jnp.float32),
        mesh=mesh,
        scratch_types=[
            pltpu.VMEM((nch, CHUNK), jnp.int32),
            pltpu.VMEM((CHUNK,), jnp.float32),
            pltpu.VMEM((rows,), jnp.float32),
            pltpu.VMEM_SHARED((nodes_pad,), jnp.float32),
            pltpu.SemaphoreType.DMA,
        ],
        name="sc_degree_hist",
    )(dstp)


def _agg_body(n, nch, g, srcp, dstp, out, src_v, dst_v, gbuf, acc, sem):
    del sem
    c = lax.axis_index("c")
    s = lax.axis_index("s")
    col = pl.multiple_of(c * CHUNK, CHUNK)
    rpt = n // NS
    rchunk = CHUNK

    def seed(k, _):
        r0 = s * rpt + k * rchunk
        pltpu.sync_copy(g.at[pl.ds(r0, rchunk), pl.ds(col, CHUNK)],
                        acc.at[pl.ds(r0, rchunk)])
        return 0

    lax.fori_loop(0, rpt // rchunk, seed, 0)

    pltpu.sync_copy(srcp.at[s], src_v)
    pltpu.sync_copy(dstp.at[s], dst_v)
    plsc.subcore_barrier()

    def edge_chunk(j, _):
        pltpu.sync_copy(g.at[src_v.at[j], pl.ds(col, CHUNK)], gbuf)
        pltpu.sync_copy(gbuf, acc.at[dst_v.at[j]], add=True)
        return 0

    lax.fori_loop(0, nch, edge_chunk, 0)
    plsc.subcore_barrier()

    def drain(k, _):
        r0 = s * rpt + k * rchunk
        pltpu.sync_copy(acc.at[pl.ds(r0, rchunk)],
                        out.at[pl.ds(r0, rchunk), pl.ds(col, CHUNK)])
        return 0

    lax.fori_loop(0, rpt // rchunk, drain, 0)


def _aggregate(g, srcp, dstp):
    np_rows = g.shape[0]
    ns, nch, _ = srcp.shape
    assert ns == NS and np_rows % (NS * CHUNK) == 0
    mesh = plsc.VectorSubcoreMesh(core_axis_name="c", subcore_axis_name="s")
    return pl.kernel(
        functools.partial(_agg_body, np_rows, nch),
        out_type=jax.ShapeDtypeStruct((np_rows, NC * CHUNK), jnp.float32),
        mesh=mesh,
        scratch_types=[
            pltpu.VMEM((nch, CHUNK), jnp.int32),
            pltpu.VMEM((nch, CHUNK), jnp.int32),
            pltpu.VMEM((CHUNK, CHUNK), jnp.float32),
            pltpu.VMEM_SHARED((np_rows, CHUNK), jnp.float32),
            pltpu.SemaphoreType.DMA,
        ],
        name="sc_edge_aggregate",
    )(g, srcp, dstp)


def _mm_body(relu_in, deg_ref, h_ref, w_ref, b_ref, g_ref):
    dis = lax.rsqrt(deg_ref[...] + 1.0)
    h = h_ref[...]
    if relu_in:
        h = jnp.maximum(h * dis, 0.0)
    acc = jnp.dot(h, w_ref[...], preferred_element_type=jnp.float32)
    g_ref[...] = (acc + b_ref[...]) * dis


def _matmul_scaled(h, w, b, deg2d, relu_in, block_rows=1024):
    n, d = h.shape
    _, hdim = w.shape
    grid = (n // block_rows,)
    return pl.pallas_call(
        functools.partial(_mm_body, relu_in),
        grid=grid,
        in_specs=[
            pl.BlockSpec((block_rows, 1), lambda i: (i, 0)),
            pl.BlockSpec((block_rows, d), lambda i: (i, 0)),
            pl.BlockSpec((d, hdim), lambda i: (0, 0)),
            pl.BlockSpec((1, hdim), lambda i: (0, 0)),
        ],
        out_specs=pl.BlockSpec((block_rows, hdim), lambda i: (i, 0)),
        out_shape=jax.ShapeDtypeStruct((n, hdim), jnp.float32),
        name="tc_matmul_scaled",
    )(deg2d, h, w, b.reshape(1, hdim))


def _relu_body(deg_ref, a_ref, o_ref):
    dis = lax.rsqrt(deg_ref[...] + 1.0)
    o_ref[...] = jnp.maximum(a_ref[...] * dis, 0.0)


def _relu_scale(a, deg2d, block_rows=1024):
    n, hdim = a.shape
    return pl.pallas_call(
        _relu_body,
        grid=(n // block_rows,),
        in_specs=[
            pl.BlockSpec((block_rows, 1), lambda i: (i, 0)),
            pl.BlockSpec((block_rows, hdim), lambda i: (i, 0)),
        ],
        out_specs=pl.BlockSpec((block_rows, hdim), lambda i: (i, 0)),
        out_shape=jax.ShapeDtypeStruct((n, hdim), jnp.float32),
        name="tc_relu_scale",
    )(deg2d, a)


def kernel(x, edge_index, W1, b1, W2, b2):
    n, d = x.shape
    e = edge_index.shape[1]
    ept = e // NS
    nch = pl.cdiv(ept, CHUNK)
    ept_pad = nch * CHUNK
    np_rows = pl.cdiv(n, NS * CHUNK) * NS * CHUNK

    src = edge_index[0].reshape(NS, ept)
    dst = edge_index[1].reshape(NS, ept)
    pad = ((0, 0), (0, ept_pad - ept))
    srcp = jnp.pad(src, pad).reshape(NS, nch, CHUNK)
    dstp = jnp.pad(dst, pad, constant_values=n).reshape(NS, nch, CHUNK)

    deg_raw = _degree_hist(dstp, np_rows)
    deg2d = deg_raw.reshape(np_rows, 1)
    xp = jnp.pad(x, ((0, np_rows - n), (0, 0)))

    g1 = _matmul_scaled(xp, W1, b1, deg2d, relu_in=False)
    a1 = _aggregate(g1, srcp, dstp)
    g2 = _matmul_scaled(a1, W2, b2, deg2d, relu_in=True)
    a2 = _aggregate(g2, srcp, dstp)
    return _relu_scale(a2, deg2d)[:n]


# speedup vs baseline: 1.0321x; 1.0062x over previous
import functools

import jax
import jax.numpy as jnp
from jax import lax
from jax.experimental import pallas as pl
from jax.experimental.pallas import tpu as pltpu
from jax.experimental.pallas import tpu_sc as plsc

NC = 2
NS = 16
CHUNK = 128


def _hist_body(nodes_pad, nch, dstp, deg, dst_v, ones_v, cnt_v, hist, sem):
    del sem
    c = lax.axis_index("c")
    s = lax.axis_index("s")
    rows = nodes_pad // NS

    def init_ones(i, _):
        ones_v[pl.ds(i * 16, 16)] = jnp.ones((16,), jnp.float32)
        return 0

    def init_zero(i, _):
        cnt_v[pl.ds(i * 16, 16)] = jnp.zeros((16,), jnp.float32)
        return 0

    lax.fori_loop(0, CHUNK // 16, init_ones, 0)
    lax.fori_loop(0, rows // 16, init_zero, 0)
    pltpu.sync_copy(cnt_v, hist.at[pl.ds(s * rows, rows)])
    plsc.subcore_barrier()

    pltpu.sync_copy(dstp.at[s], dst_v)

    def scatter(j, _):
        pltpu.sync_copy(ones_v, hist.at[dst_v.at[j]], add=True)
        return 0

    lax.fori_loop(0, nch, scatter, 0)
    plsc.subcore_barrier()

    @pl.when(c == 0)
    def _drain():
        pltpu.sync_copy(hist.at[pl.ds(s * rows, rows)], cnt_v)
        pltpu.sync_copy(cnt_v, deg.at[pl.ds(s * rows, rows)])


def _degree_hist(dstp, nodes_pad):
    ns, nch, _ = dstp.shape
    assert ns == NS
    rows = nodes_pad // NS
    mesh = plsc.VectorSubcoreMesh(core_axis_name="c", subcore_axis_name="s")
    return pl.kernel(
        functools.partial(_hist_body, nodes_pad, nch),
        out_type=jax.ShapeDtypeStruct((nodes_pad,), jnp.float32),
        mesh=mesh,
        scratch_types=[
            pltpu.VMEM((nch, CHUNK), jnp.int32),
            pltpu.VMEM((CHUNK,), jnp.float32),
            pltpu.VMEM((rows,), jnp.float32),
            pltpu.VMEM_SHARED((nodes_pad,), jnp.float32),
            pltpu.SemaphoreType.DMA,
        ],
        name="sc_degree_hist",
    )(dstp)


def _agg_body(n, nch, g, srcp, dstp, out, src_v, dst_v, gbuf, acc, sem):
    del sem
    c = lax.axis_index("c")
    s = lax.axis_index("s")
    col = pl.multiple_of(c * CHUNK, CHUNK)
    rpt = n // NS
    rchunk = CHUNK

    def seed(k, _):
        r0 = s * rpt + k * rchunk
        pltpu.sync_copy(g.at[pl.ds(r0, rchunk), pl.ds(col, CHUNK)],
                        acc.at[pl.ds(r0, rchunk)])
        return 0

    lax.fori_loop(0, rpt // rchunk, seed, 0)

    pltpu.sync_copy(srcp.at[s], src_v)
    pltpu.sync_copy(dstp.at[s], dst_v)
    plsc.subcore_barrier()

    def edge_chunk(j, _):
        pltpu.sync_copy(g.at[src_v.at[j], pl.ds(col, CHUNK)], gbuf)
        pltpu.sync_copy(gbuf, acc.at[dst_v.at[j]], add=True)
        return 0

    lax.fori_loop(0, nch, edge_chunk, 0)
    plsc.subcore_barrier()

    def drain(k, _):
        r0 = s * rpt + k * rchunk
        pltpu.sync_copy(acc.at[pl.ds(r0, rchunk)],
                        out.at[pl.ds(r0, rchunk), pl.ds(col, CHUNK)])
        return 0

    lax.fori_loop(0, rpt // rchunk, drain, 0)


def _aggregate(g, srcp, dstp):
    np_rows = g.shape[0]
    ns, nch, _ = srcp.shape
    assert ns == NS and np_rows % (NS * CHUNK) == 0
    mesh = plsc.VectorSubcoreMesh(core_axis_name="c", subcore_axis_name="s")
    return pl.kernel(
        functools.partial(_agg_body, np_rows, nch),
        out_type=jax.ShapeDtypeStruct((np_rows, NC * CHUNK), jnp.float32),
        mesh=mesh,
        scratch_types=[
            pltpu.VMEM((nch, CHUNK), jnp.int32),
            pltpu.VMEM((nch, CHUNK), jnp.int32),
            pltpu.VMEM((CHUNK, CHUNK), jnp.float32),
            pltpu.VMEM_SHARED((np_rows, CHUNK), jnp.float32),
            pltpu.SemaphoreType.DMA,
        ],
        name="sc_edge_aggregate",
    )(g, srcp, dstp)


def _mm_body(relu_in, deg_ref, h_ref, w_ref, b_ref, g_ref):
    dis = lax.rsqrt(deg_ref[...] + 1.0)
    h = h_ref[...]
    if relu_in:
        h = jnp.maximum(h * dis, 0.0)
    acc = jnp.dot(h, w_ref[...], preferred_element_type=jnp.float32)
    g_ref[...] = (acc + b_ref[...]) * dis


def _matmul_scaled(h, w, b, deg2d, relu_in, block_rows=1024):
    n, d = h.shape
    _, hdim = w.shape
    grid = (n // block_rows,)
    return pl.pallas_call(
        functools.partial(_mm_body, relu_in),
        grid=grid,
        in_specs=[
            pl.BlockSpec((block_rows, 1), lambda i: (i, 0)),
            pl.BlockSpec((block_rows, d), lambda i: (i, 0)),
            pl.BlockSpec((d, hdim), lambda i: (0, 0)),
            pl.BlockSpec((1, hdim), lambda i: (0, 0)),
        ],
        out_specs=pl.BlockSpec((block_rows, hdim), lambda i: (i, 0)),
        out_shape=jax.ShapeDtypeStruct((n, hdim), jnp.float32),
        name="tc_matmul_scaled",
    )(deg2d, h, w, b.reshape(1, hdim))


def _relu_body(deg_ref, a_ref, o_ref):
    dis = lax.rsqrt(deg_ref[...] + 1.0)
    o_ref[...] = jnp.maximum(a_ref[...] * dis, 0.0)


def _relu_scale(a, deg2d, n_out, block_rows=1000):
    hdim = a.shape[1]
    assert n_out % block_rows == 0
    return pl.pallas_call(
        _relu_body,
        grid=(n_out // block_rows,),
        in_specs=[
            pl.BlockSpec((block_rows, 1), lambda i: (i, 0)),
            pl.BlockSpec((block_rows, hdim), lambda i: (i, 0)),
        ],
        out_specs=pl.BlockSpec((block_rows, hdim), lambda i: (i, 0)),
        out_shape=jax.ShapeDtypeStruct((n_out, hdim), jnp.float32),
        name="tc_relu_scale",
    )(deg2d, a)


def kernel(x, edge_index, W1, b1, W2, b2):
    n, d = x.shape
    e = edge_index.shape[1]
    ept = e // NS
    nch = pl.cdiv(ept, CHUNK)
    ept_pad = nch * CHUNK
    np_rows = pl.cdiv(n, NS * CHUNK) * NS * CHUNK

    src = edge_index[0].reshape(NS, ept)
    dst = edge_index[1].reshape(NS, ept)
    pad = ((0, 0), (0, ept_pad - ept))
    srcp = jnp.pad(src, pad).reshape(NS, nch, CHUNK)
    dstp = jnp.pad(dst, pad, constant_values=n).reshape(NS, nch, CHUNK)

    deg_raw = _degree_hist(dstp, np_rows)
    deg2d = deg_raw.reshape(np_rows, 1)
    xp = jnp.pad(x, ((0, np_rows - n), (0, 0)))

    g1 = _matmul_scaled(xp, W1, b1, deg2d, relu_in=False)
    a1 = _aggregate(g1, srcp, dstp)
    g2 = _matmul_scaled(a1, W2, b2, deg2d, relu_in=True)
    a2 = _aggregate(g2, srcp, dstp)
    return _relu_scale(a2, deg2d, n)


# speedup vs baseline: 1.0393x; 1.0070x over previous
import functools

import jax
import jax.numpy as jnp
from jax import lax
from jax.experimental import pallas as pl
from jax.experimental.pallas import tpu as pltpu
from jax.experimental.pallas import tpu_sc as plsc

NC = 2
NS = 16
CHUNK = 128


def _hist_body(nodes_pad, nch, dstp, deg, dst_v, ones_v, cnt_v, hist, sem):
    del sem
    c = lax.axis_index("c")
    s = lax.axis_index("s")
    rows = nodes_pad // NS

    def init_ones(i, _):
        ones_v[pl.ds(i * 16, 16)] = jnp.ones((16,), jnp.float32)
        return 0

    def init_zero(i, _):
        cnt_v[pl.ds(i * 16, 16)] = jnp.zeros((16,), jnp.float32)
        return 0

    lax.fori_loop(0, CHUNK // 16, init_ones, 0)
    lax.fori_loop(0, rows // 16, init_zero, 0)
    pltpu.sync_copy(cnt_v, hist.at[pl.ds(s * rows, rows)])
    plsc.subcore_barrier()

    pltpu.sync_copy(dstp.at[s], dst_v)

    def scatter(j, _):
        pltpu.sync_copy(ones_v, hist.at[dst_v.at[j]], add=True)
        return 0

    lax.fori_loop(0, nch, scatter, 0)
    plsc.subcore_barrier()

    @pl.when(c == 0)
    def _drain():
        pltpu.sync_copy(hist.at[pl.ds(s * rows, rows)], cnt_v)
        pltpu.sync_copy(cnt_v, deg.at[pl.ds(s * rows, rows)])


def _degree_hist(dstp, nodes_pad):
    ns, nch, _ = dstp.shape
    assert ns == NS
    rows = nodes_pad // NS
    mesh = plsc.VectorSubcoreMesh(core_axis_name="c", subcore_axis_name="s")
    return pl.kernel(
        functools.partial(_hist_body, nodes_pad, nch),
        out_type=jax.ShapeDtypeStruct((nodes_pad,), jnp.float32),
        mesh=mesh,
        scratch_types=[
            pltpu.VMEM((nch, CHUNK), jnp.int32),
            pltpu.VMEM((CHUNK,), jnp.float32),
            pltpu.VMEM((rows,), jnp.float32),
            pltpu.VMEM_SHARED((nodes_pad,), jnp.float32),
            pltpu.SemaphoreType.DMA,
        ],
        name="sc_degree_hist",
    )(dstp)


def _agg_body(n, nch, g, srcp, dstp, out, src_v, dst_v, gbuf, acc, sem):
    del sem
    c = lax.axis_index("c")
    s = lax.axis_index("s")
    col = pl.multiple_of(c * CHUNK, CHUNK)
    rpt = n // NS
    rchunk = CHUNK

    def seed(k, _):
        r0 = s * rpt + k * rchunk
        pltpu.sync_copy(g.at[pl.ds(r0, rchunk), pl.ds(col, CHUNK)],
                        acc.at[pl.ds(r0, rchunk)])
        return 0

    lax.fori_loop(0, rpt // rchunk, seed, 0)

    pltpu.sync_copy(srcp.at[s], src_v)
    pltpu.sync_copy(dstp.at[s], dst_v)
    plsc.subcore_barrier()

    def edge_chunk(j, _):
        pltpu.sync_copy(g.at[src_v.at[j], pl.ds(col, CHUNK)], gbuf)
        pltpu.sync_copy(gbuf, acc.at[dst_v.at[j]], add=True)
        return 0

    lax.fori_loop(0, nch, edge_chunk, 0)
    plsc.subcore_barrier()

    def drain(k, _):
        r0 = s * rpt + k * rchunk
        pltpu.sync_copy(acc.at[pl.ds(r0, rchunk)],
                        out.at[pl.ds(r0, rchunk), pl.ds(col, CHUNK)])
        return 0

    lax.fori_loop(0, rpt // rchunk, drain, 0)


def _aggregate(g, srcp, dstp):
    np_rows = g.shape[0]
    ns, nch, _ = srcp.shape
    assert ns == NS and np_rows % (NS * CHUNK) == 0
    mesh = plsc.VectorSubcoreMesh(core_axis_name="c", subcore_axis_name="s")
    return pl.kernel(
        functools.partial(_agg_body, np_rows, nch),
        out_type=jax.ShapeDtypeStruct((np_rows, NC * CHUNK), jnp.float32),
        mesh=mesh,
        scratch_types=[
            pltpu.VMEM((nch, CHUNK), jnp.int32),
            pltpu.VMEM((nch, CHUNK), jnp.int32),
            pltpu.VMEM((CHUNK, CHUNK), jnp.float32),
            pltpu.VMEM_SHARED((np_rows, CHUNK), jnp.float32),
            pltpu.SemaphoreType.DMA,
        ],
        name="sc_edge_aggregate",
    )(g, srcp, dstp)


def _mm_body(relu_in, deg_ref, h_ref, w_ref, b_ref, g_ref):
    dis = lax.rsqrt(deg_ref[...] + 1.0)
    h = h_ref[...]
    if relu_in:
        h = jnp.maximum(h * dis, 0.0)
    acc = jnp.dot(h, w_ref[...], preferred_element_type=jnp.float32)
    g_ref[...] = (acc + b_ref[...]) * dis


def _matmul_scaled(h, w, b, deg2d, relu_in, out_rows, block_rows=1024):
    d = h.shape[1]
    _, hdim = w.shape
    assert out_rows % block_rows == 0
    return pl.pallas_call(
        functools.partial(_mm_body, relu_in),
        grid=(out_rows // block_rows,),
        in_specs=[
            pl.BlockSpec((block_rows, 1), lambda i: (i, 0)),
            pl.BlockSpec((block_rows, d), lambda i: (i, 0)),
            pl.BlockSpec((d, hdim), lambda i: (0, 0)),
            pl.BlockSpec((1, hdim), lambda i: (0, 0)),
        ],
        out_specs=pl.BlockSpec((block_rows, hdim), lambda i: (i, 0)),
        out_shape=jax.ShapeDtypeStruct((out_rows, hdim), jnp.float32),
        name="tc_matmul_scaled",
    )(deg2d, h, w, b.reshape(1, hdim))


def _relu_body(deg_ref, a_ref, o_ref):
    dis = lax.rsqrt(deg_ref[...] + 1.0)
    o_ref[...] = jnp.maximum(a_ref[...] * dis, 0.0)


def _relu_scale(a, deg2d, n_out, block_rows=1000):
    hdim = a.shape[1]
    assert n_out % block_rows == 0
    return pl.pallas_call(
        _relu_body,
        grid=(n_out // block_rows,),
        in_specs=[
            pl.BlockSpec((block_rows, 1), lambda i: (i, 0)),
            pl.BlockSpec((block_rows, hdim), lambda i: (i, 0)),
        ],
        out_specs=pl.BlockSpec((block_rows, hdim), lambda i: (i, 0)),
        out_shape=jax.ShapeDtypeStruct((n_out, hdim), jnp.float32),
        name="tc_relu_scale",
    )(deg2d, a)


def kernel(x, edge_index, W1, b1, W2, b2):
    n, d = x.shape
    e = edge_index.shape[1]
    ept = e // NS
    nch = pl.cdiv(ept, CHUNK)
    ept_pad = nch * CHUNK
    np_rows = pl.cdiv(n, NS * CHUNK) * NS * CHUNK

    src = edge_index[0].reshape(NS, ept)
    dst = edge_index[1].reshape(NS, ept)
    pad = ((0, 0), (0, ept_pad - ept))
    srcp = jnp.pad(src, pad).reshape(NS, nch, CHUNK)
    dstp = jnp.pad(dst, pad, constant_values=n).reshape(NS, nch, CHUNK)

    deg_raw = _degree_hist(dstp, np_rows)
    deg2d = deg_raw.reshape(np_rows, 1)

    g1 = _matmul_scaled(x, W1, b1, deg2d, relu_in=False, out_rows=np_rows)
    a1 = _aggregate(g1, srcp, dstp)
    g2 = _matmul_scaled(a1, W2, b2, deg2d, relu_in=True, out_rows=np_rows)
    a2 = _aggregate(g2, srcp, dstp)
    return _relu_scale(a2, deg2d, n)


# speedup vs baseline: 1.0402x; 1.0008x over previous
import functools

import jax
import jax.numpy as jnp
from jax import lax
from jax.experimental import pallas as pl
from jax.experimental.pallas import tpu as pltpu
from jax.experimental.pallas import tpu_sc as plsc

NC = 2
NS = 16
CHUNK = 128


def _hist_body(nodes_pad, nch, dstp, deg, dst_v, ones_v, cnt_v, hist, sem):
    del sem
    c = lax.axis_index("c")
    s = lax.axis_index("s")
    rows = nodes_pad // NS

    def init_ones(i, _):
        ones_v[pl.ds(i * 16, 16)] = jnp.ones((16,), jnp.float32)
        return 0

    def init_zero(i, _):
        cnt_v[pl.ds(i * 16, 16)] = jnp.zeros((16,), jnp.float32)
        return 0

    lax.fori_loop(0, CHUNK // 16, init_ones, 0)
    lax.fori_loop(0, rows // 16, init_zero, 0)
    pltpu.sync_copy(cnt_v, hist.at[pl.ds(s * rows, rows)])
    plsc.subcore_barrier()

    pltpu.sync_copy(dstp.at[s], dst_v)

    def scatter(j, _):
        pltpu.sync_copy(ones_v, hist.at[dst_v.at[j]], add=True)
        return 0

    lax.fori_loop(0, nch, scatter, 0)
    plsc.subcore_barrier()

    @pl.when(c == 0)
    def _drain():
        pltpu.sync_copy(hist.at[pl.ds(s * rows, rows)], cnt_v)
        pltpu.sync_copy(cnt_v, deg.at[pl.ds(s * rows, rows)])


def _degree_hist(dstp, nodes_pad):
    ns, nch, _ = dstp.shape
    assert ns == NS
    rows = nodes_pad // NS
    mesh = plsc.VectorSubcoreMesh(core_axis_name="c", subcore_axis_name="s")
    return pl.kernel(
        functools.partial(_hist_body, nodes_pad, nch),
        out_type=jax.ShapeDtypeStruct((nodes_pad,), jnp.float32),
        mesh=mesh,
        scratch_types=[
            pltpu.VMEM((nch, CHUNK), jnp.int32),
            pltpu.VMEM((CHUNK,), jnp.float32),
            pltpu.VMEM((rows,), jnp.float32),
            pltpu.VMEM_SHARED((nodes_pad,), jnp.float32),
            pltpu.SemaphoreType.DMA,
        ],
        name="sc_degree_hist",
    )(dstp)


def _agg_body(n, nch, g, srcp, dstp, out, src_v, dst_v, gbuf, acc, sem):
    del sem
    c = lax.axis_index("c")
    s = lax.axis_index("s")
    col = pl.multiple_of(c * CHUNK, CHUNK)
    rpt = n // NS
    rchunk = CHUNK

    def seed(k, _):
        r0 = s * rpt + k * rchunk
        pltpu.sync_copy(g.at[pl.ds(r0, rchunk), pl.ds(col, CHUNK)],
                        acc.at[pl.ds(r0, rchunk)])
        return 0

    lax.fori_loop(0, rpt // rchunk, seed, 0)

    pltpu.sync_copy(srcp.at[s], src_v)
    pltpu.sync_copy(dstp.at[s], dst_v)
    plsc.subcore_barrier()

    def edge_chunk(j, _):
        pltpu.sync_copy(g.at[src_v.at[j], pl.ds(col, CHUNK)], gbuf)
        pltpu.sync_copy(gbuf, acc.at[dst_v.at[j]], add=True)
        return 0

    lax.fori_loop(0, nch, edge_chunk, 0)
    plsc.subcore_barrier()

    def drain(k, _):
        r0 = s * rpt + k * rchunk
        pltpu.sync_copy(acc.at[pl.ds(r0, rchunk)],
                        out.at[pl.ds(r0, rchunk), pl.ds(col, CHUNK)])
        return 0

    lax.fori_loop(0, rpt // rchunk, drain, 0)


def _aggregate(g, srcp, dstp):
    np_rows = g.shape[0]
    ns, nch, _ = srcp.shape
    assert ns == NS and np_rows % (NS * CHUNK) == 0
    mesh = plsc.VectorSubcoreMesh(core_axis_name="c", subcore_axis_name="s")
    return pl.kernel(
        functools.partial(_agg_body, np_rows, nch),
        out_type=jax.ShapeDtypeStruct((np_rows, NC * CHUNK), jnp.float32),
        mesh=mesh,
        scratch_types=[
            pltpu.VMEM((nch, CHUNK), jnp.int32),
            pltpu.VMEM((nch, CHUNK), jnp.int32),
            pltpu.VMEM((CHUNK, CHUNK), jnp.float32),
            pltpu.VMEM_SHARED((np_rows, CHUNK), jnp.float32),
            pltpu.SemaphoreType.DMA,
        ],
        name="sc_edge_aggregate",
    )(g, srcp, dstp)


def _mm_body(relu_in, deg_ref, h_ref, w_ref, b_ref, g_ref):
    dis = lax.rsqrt(deg_ref[...] + 1.0)
    h = h_ref[...]
    if relu_in:
        h = jnp.maximum(h * dis, 0.0)
    acc = jnp.dot(h, w_ref[...], preferred_element_type=jnp.float32)
    g_ref[...] = (acc + b_ref[...]) * dis


def _matmul_scaled(h, w, b, deg2d, relu_in, block_rows=1024):
    n, d = h.shape
    _, hdim = w.shape
    return pl.pallas_call(
        functools.partial(_mm_body, relu_in),
        grid=(n // block_rows,),
        in_specs=[
            pl.BlockSpec((block_rows, 1), lambda i: (i, 0)),
            pl.BlockSpec((block_rows, d), lambda i: (i, 0)),
            pl.BlockSpec((d, hdim), lambda i: (0, 0)),
            pl.BlockSpec((1, hdim), lambda i: (0, 0)),
        ],
        out_specs=pl.BlockSpec((block_rows, hdim), lambda i: (i, 0)),
        out_shape=jax.ShapeDtypeStruct((n, hdim), jnp.float32),
        name="tc_matmul_scaled",
    )(deg2d, h, w, b.reshape(1, hdim))


def _relu_body(deg_ref, a_ref, o_ref):
    dis = lax.rsqrt(deg_ref[...] + 1.0)
    o_ref[...] = jnp.maximum(a_ref[...] * dis, 0.0)


def _relu_scale(a, deg2d, n_out, block_rows=1000):
    hdim = a.shape[1]
    assert n_out % block_rows == 0
    return pl.pallas_call(
        _relu_body,
        grid=(n_out // block_rows,),
        in_specs=[
            pl.BlockSpec((block_rows, 1), lambda i: (i, 0)),
            pl.BlockSpec((block_rows, hdim), lambda i: (i, 0)),
        ],
        out_specs=pl.BlockSpec((block_rows, hdim), lambda i: (i, 0)),
        out_shape=jax.ShapeDtypeStruct((n_out, hdim), jnp.float32),
        name="tc_relu_scale",
    )(deg2d, a)


def kernel(x, edge_index, W1, b1, W2, b2):
    n, d = x.shape
    e = edge_index.shape[1]
    ept = e // NS
    nch = pl.cdiv(ept, CHUNK)
    ept_pad = nch * CHUNK
    np_rows = pl.cdiv(n, NS * CHUNK) * NS * CHUNK

    src = edge_index[0].reshape(NS, ept)
    dst = edge_index[1].reshape(NS, ept)
    pad = ((0, 0), (0, ept_pad - ept))
    srcp = jnp.pad(src, pad).reshape(NS, nch, CHUNK)
    dstp = jnp.pad(dst, pad, constant_values=n).reshape(NS, nch, CHUNK)

    deg_raw = _degree_hist(dstp, np_rows)
    deg2d = deg_raw.reshape(np_rows, 1)
    xp = jnp.pad(x, ((0, np_rows - n), (0, 0)))

    g1 = _matmul_scaled(xp, W1, b1, deg2d, relu_in=False)
    a1 = _aggregate(g1, srcp, dstp)
    g2 = _matmul_scaled(a1, W2, b2, deg2d, relu_in=True)
    a2 = _aggregate(g2, srcp, dstp)
    return _relu_scale(a2, deg2d, n)
